# Initial kernel scaffold; baseline (speedup 1.0000x reference)
import jax
import jax.numpy as jnp
from jax.experimental import pallas as pl


def kernel(x, analysis_window):
    raise NotImplementedError("write your pallas kernel here")


# speedup vs baseline: 18.7436x; 18.7436x over previous
import jax
import jax.numpy as jnp
from jax.experimental import pallas as pl

_HOP = 256
_SEG = 512


def _frames_kernel(x_ref, w_ref, o_ref):
    nf = o_ref.shape[1]
    a = x_ref[0, 0:nf, :]
    b = x_ref[0, 1:nf + 1, :]
    o_ref[0, :, 0:_HOP] = a * w_ref[0, :]
    o_ref[0, :, _HOP:_SEG] = b * w_ref[1, :]


def kernel(x, analysis_window):
    batch, num_samples = x.shape
    num_chunks = num_samples // _HOP
    num_frames = (num_samples - _SEG) // _HOP + 1

    x3 = x.reshape(batch, num_chunks, _HOP)
    w2 = analysis_window.reshape(2, _HOP)

    return pl.pallas_call(
        _frames_kernel,
        grid=(batch,),
        in_specs=[
            pl.BlockSpec((1, num_chunks, _HOP), lambda b: (b, 0, 0)),
            pl.BlockSpec((2, _HOP), lambda b: (0, 0)),
        ],
        out_specs=pl.BlockSpec((1, num_frames, _SEG), lambda b: (b, 0, 0)),
        out_shape=jax.ShapeDtypeStruct((batch, num_frames, _SEG), x.dtype),
    )(x3, w2)
